# baseline (speedup 1.0000x reference)
import jax
import jax.numpy as jnp
from jax.experimental import pallas as pl

_TM = 1024


def _router_kernel(x_ref, w_ref, out_ref, z_ref):
    i = pl.program_id(0)
    logits = jnp.dot(x_ref[...], w_ref[...], preferred_element_type=jnp.float32)
    m = jnp.max(logits, axis=-1, keepdims=True)
    e = jnp.exp(logits - m)
    s = jnp.sum(e, axis=-1, keepdims=True)
    out_ref[...] = jnp.concatenate([e / s, logits], axis=1)
    lse = m + jnp.log(s)
    part = jnp.sum(lse * lse, keepdims=True)

    @pl.when(i == 0)
    def _init():
        z_ref[...] = part

    @pl.when(i != 0)
    def _acc():
        z_ref[...] += part


def kernel(token_inputs, W, expert_capacity):
    g, t, h = token_inputs.shape
    e = W.shape[1]
    n = g * t
    x = token_inputs.reshape(n, h)
    packed, z = pl.pallas_call(
        _router_kernel,
        grid=(n // _TM,),
        in_specs=[
            pl.BlockSpec((_TM, h), lambda i: (i, 0)),
            pl.BlockSpec((h, e), lambda i: (0, 0)),
        ],
        out_specs=[
            pl.BlockSpec((_TM, 2 * e), lambda i: (i, 0)),
            pl.BlockSpec((1, 1), lambda i: (0, 0)),
        ],
        out_shape=[
            jax.ShapeDtypeStruct((n, 2 * e), jnp.float32),
            jax.ShapeDtypeStruct((1, 1), jnp.float32),
        ],
    )(x, W)
    z_loss = z[0, 0] / n
    probs = packed[:, :e].reshape(g, t, e)
    logits = packed[:, e:].reshape(g, t, e)
    return probs, logits, z_loss


# speedup vs baseline: 1.3258x; 1.3258x over previous
import jax
import jax.numpy as jnp
from jax.experimental import pallas as pl

_TM = 1024


def _router_kernel(x_ref, w_ref, p_ref, l_ref, z_ref):
    i = pl.program_id(0)
    logits = jax.lax.dot_general(
        w_ref[...], x_ref[...], (((0,), (1,)), ((), ())),
        preferred_element_type=jnp.float32)
    m = jnp.max(logits, axis=0, keepdims=True)
    e = jnp.exp(logits - m)
    s = jnp.sum(e, axis=0, keepdims=True)
    p_ref[...] = e / s
    l_ref[...] = logits
    lse = m + jnp.log(s)
    part = jnp.sum(lse * lse, keepdims=True)

    @pl.when(i == 0)
    def _init():
        z_ref[...] = part

    @pl.when(i != 0)
    def _acc():
        z_ref[...] += part


def kernel(token_inputs, W, expert_capacity):
    g, t, h = token_inputs.shape
    e = W.shape[1]
    n = g * t
    x = token_inputs.reshape(n, h)
    probsT, logitsT, z = pl.pallas_call(
        _router_kernel,
        grid=(n // _TM,),
        in_specs=[
            pl.BlockSpec((_TM, h), lambda i: (i, 0)),
            pl.BlockSpec((h, e), lambda i: (0, 0)),
        ],
        out_specs=[
            pl.BlockSpec((e, _TM), lambda i: (0, i)),
            pl.BlockSpec((e, _TM), lambda i: (0, i)),
            pl.BlockSpec((1, 1), lambda i: (0, 0)),
        ],
        out_shape=[
            jax.ShapeDtypeStruct((e, n), jnp.float32),
            jax.ShapeDtypeStruct((e, n), jnp.float32),
            jax.ShapeDtypeStruct((1, 1), jnp.float32),
        ],
    )(x, W)
    z_loss = z[0, 0] / n
    probs = probsT.T.reshape(g, t, e)
    logits = logitsT.T.reshape(g, t, e)
    return probs, logits, z_loss
